# baseline (speedup 1.0000x reference)
import functools

import jax
import jax.numpy as jnp
from jax import lax
from jax.experimental import pallas as pl
from jax.experimental.pallas import tpu as pltpu
from jax.experimental.pallas import tpu_sc as plsc

N = 10000
E = 320000
D = 128
HD = 64
R = 4
EC = 12
ECP = 16

NC = 2
NS = 16
NW = NC * NS
CH = 125
NCH = 80
EPT = NCH * CH
NACC = 10112
ZR = NACC // NS
BLK = 1000


@functools.lru_cache(maxsize=None)
def _make_segsum_sc():
    mesh = plsc.VectorSubcoreMesh(core_axis_name="c", subcore_axis_name="s")

    @functools.partial(
        pl.kernel,
        mesh=mesh,
        compiler_params=pltpu.CompilerParams(use_tc_tiling_on_sc=False),
        out_type=jax.ShapeDtypeStruct((2, NC, NACC, HD), jnp.float32),
        scratch_types=[
            pltpu.VMEM((NCH, CH), jnp.int32),
            pltpu.VMEM((NCH, CH), jnp.int32),
            pltpu.VMEM((CH, HD), jnp.float32),
            pltpu.VMEM((CH, HD), jnp.float32),
            pltpu.VMEM_SHARED((NACC, HD), jnp.float32),
            pltpu.VMEM_SHARED((N, HD), jnp.float32),
            pltpu.SemaphoreType.DMA,
            pltpu.SemaphoreType.DMA,
        ],
    )
    def _segsum_sc(x_hbm, src_hbm, dst_hbm, zeros_hbm, out_hbm,
                   src_v, dst_v, rows0, rows1, acc, table, sem0, sem1):
        c = lax.axis_index("c")
        s = lax.axis_index("s")
        wid = s * NC + c

        with jax.named_scope("sc_idx"):
            pltpu.sync_copy(src_hbm.at[wid], src_v)
            pltpu.sync_copy(dst_hbm.at[wid], dst_v)

        for h in range(2):
            with jax.named_scope("sc_stage"):
                pltpu.sync_copy(zeros_hbm, acc.at[pl.ds(s * ZR, ZR)])

                @pl.when(s < 10)
                def _stage_table():
                    pltpu.sync_copy(x_hbm.at[h, pl.ds(s * 1000, 1000)],
                                    table.at[pl.ds(s * 1000, 1000)])

                plsc.subcore_barrier()

            last = NCH - 1
            with jax.named_scope("sc_mainloop"):
                pltpu.async_copy(table.at[src_v.at[0]], rows0, sem0)

                def body(i, carry):
                    j = 2 * i
                    pltpu.async_copy(
                        table.at[src_v.at[jnp.minimum(j + 1, last)]],
                        rows1, sem1)
                    pltpu.make_async_copy(
                        table.at[src_v.at[0]], rows0, sem0).wait()
                    pltpu.sync_copy(rows0, acc.at[dst_v.at[j]], add=True)
                    pltpu.async_copy(
                        table.at[src_v.at[jnp.minimum(j + 2, last)]],
                        rows0, sem0)
                    pltpu.make_async_copy(
                        table.at[src_v.at[0]], rows1, sem1).wait()
                    pltpu.sync_copy(rows1, acc.at[dst_v.at[j + 1]], add=True)
                    return carry

                lax.fori_loop(0, NCH // 2, body, 0)
                pltpu.make_async_copy(table.at[src_v.at[0]], rows0, sem0).wait()
                plsc.subcore_barrier()

            with jax.named_scope("sc_out"):
                pltpu.sync_copy(acc.at[pl.ds(s * ZR, ZR)],
                                out_hbm.at[h, c, pl.ds(s * ZR, ZR)])

    return _segsum_sc


def _ew_body(h_ref, src_row_ref, src_col_ref, dst_row_ref, w_ref,
             beta_ref, eps_ref, W_ref, b_ref, out_ref):
    h = h_ref[...]
    csrc = src_row_ref[...]
    csrc_col = src_col_ref[...]
    cdst = dst_row_ref[...]
    w = w_ref[...]
    seg = lax.broadcasted_iota(jnp.int32, (R, ECP), 0)
    ohs = (seg == csrc)
    ohd = (seg == cdst).astype(jnp.float32)
    for l in range(2):
        nrm = jnp.sqrt(jnp.sum(h * h, axis=1, keepdims=True))
        norm_h = h / (nrm + 1e-12)
        e = beta_ref[l] * w
        m = jnp.max(jnp.where(ohs, e, -1e30), axis=1, keepdims=True)
        m = jnp.where(m < -1e29, 0.0, m)
        m_pe = jnp.sum(jnp.where(ohs, m, 0.0), axis=0, keepdims=True)
        ex = jnp.exp(e - m_pe)
        ssum = jnp.sum(jnp.where(ohs, ex, 0.0), axis=1, keepdims=True)
        s_pe = jnp.sum(jnp.where(ohs, ssum, 0.0), axis=0, keepdims=True)
        p = ex / (s_pe + 1e-16)
        gath = jnp.zeros((ECP, D), jnp.float32)
        for r in range(R):
            gath = gath + jnp.where(csrc_col == r, 1.0, 0.0) * norm_h[r:r + 1, :]
        agg = jnp.dot(ohd * p, gath,
                      preferred_element_type=jnp.float32,
                      precision=lax.Precision.HIGHEST)
        h = (1.0 + eps_ref[l]) * h + agg
        h = jnp.maximum(h, 0.0)
    ew = jnp.dot(h[0:1, :], W_ref[...],
                 preferred_element_type=jnp.float32,
                 precision=lax.Precision.HIGHEST) + b_ref[...]
    out_ref[...] = ew


def _ew_call(cl_h, src_row, src_col, dst_row, w_row, beta, eps, W, b):
    return pl.pallas_call(
        _ew_body,
        out_shape=jax.ShapeDtypeStruct((1, D), jnp.float32),
        in_specs=[
            pl.BlockSpec((R, D), lambda: (0, 0)),
            pl.BlockSpec((1, ECP), lambda: (0, 0)),
            pl.BlockSpec((ECP, 1), lambda: (0, 0)),
            pl.BlockSpec((1, ECP), lambda: (0, 0)),
            pl.BlockSpec((1, ECP), lambda: (0, 0)),
            pl.BlockSpec(memory_space=pltpu.SMEM),
            pl.BlockSpec(memory_space=pltpu.SMEM),
            pl.BlockSpec((D, D), lambda: (0, 0)),
            pl.BlockSpec((1, D), lambda: (0, 0)),
        ],
        out_specs=pl.BlockSpec((1, D), lambda: (0, 0)),
    )(cl_h, src_row, src_col, dst_row, w_row, beta, eps, W, b)


def _layer_body(x_ref, p_ref, ew_ref, W_ref, out_ref):
    x = jnp.concatenate([x_ref[0], x_ref[1]], axis=1)
    agg = jnp.concatenate([p_ref[0, 0] + p_ref[0, 1],
                           p_ref[1, 0] + p_ref[1, 1]], axis=1) * ew_ref[...]
    t = jnp.tanh(
        jnp.dot(agg + x, W_ref[...],
                preferred_element_type=jnp.float32,
                precision=lax.Precision.HIGHEST))
    out_ref[0] = t[:, :HD]
    out_ref[1] = t[:, HD:]


def _layer2_body(x_ref, p_ref, ew_ref, W_ref, lW_ref, out_ref):
    x = jnp.concatenate([x_ref[0], x_ref[1]], axis=1)
    agg = jnp.concatenate([p_ref[0, 0] + p_ref[0, 1],
                           p_ref[1, 0] + p_ref[1, 1]], axis=1) * ew_ref[...]
    t = jnp.tanh(
        jnp.dot(agg + x, W_ref[...],
                preferred_element_type=jnp.float32,
                precision=lax.Precision.HIGHEST))
    out_ref[...] = jnp.dot(t, lW_ref[...],
                           preferred_element_type=jnp.float32,
                           precision=lax.Precision.HIGHEST)


def _common_specs():
    return [
        pl.BlockSpec((2, BLK, HD), lambda i: (0, i, 0)),
        pl.BlockSpec((2, NC, BLK, HD), lambda i: (0, 0, i, 0)),
        pl.BlockSpec((1, D), lambda i: (0, 0)),
        pl.BlockSpec((D, D), lambda i: (0, 0)),
    ]


def _layer_call(x, p, ew, W):
    return pl.pallas_call(
        _layer_body,
        grid=(N // BLK,),
        out_shape=jax.ShapeDtypeStruct((2, N, HD), jnp.float32),
        in_specs=_common_specs(),
        out_specs=pl.BlockSpec((2, BLK, HD), lambda i: (0, i, 0)),
    )(x, p, ew, W)


def _layer2_call(x, p, ew, W, lW):
    return pl.pallas_call(
        _layer2_body,
        grid=(N // BLK,),
        out_shape=jax.ShapeDtypeStruct((N, D), jnp.float32),
        in_specs=_common_specs() + [pl.BlockSpec((D, D), lambda i: (0, 0))],
        out_specs=pl.BlockSpec((BLK, D), lambda i: (i, 0)),
    )(x, p, ew, W, lW)


def kernel(node_feat, edge_index, cl_h, cl_edge_index, cl_edge_w,
           beta, eps, lin_e1_W, lin_e1_b, gc_W, lin_W):
    src_r = edge_index[0].reshape(NW, NCH, CH)
    dst_r = edge_index[1].reshape(NW, NCH, CH)
    zeros = jnp.zeros((ZR, HD), jnp.float32)

    cpad = ECP - EC
    src_row = jnp.concatenate(
        [cl_edge_index[0], jnp.full((cpad,), R + 1, jnp.int32)]).reshape(1, ECP)
    src_col = src_row.reshape(ECP, 1)
    dst_row = jnp.concatenate(
        [cl_edge_index[1], jnp.full((cpad,), R + 1, jnp.int32)]).reshape(1, ECP)
    w_row = jnp.concatenate(
        [cl_edge_w, jnp.zeros((cpad,), jnp.float32)]).reshape(1, ECP)

    ew = _ew_call(cl_h, src_row, src_col, dst_row, w_row,
                  beta, eps, lin_e1_W, lin_e1_b.reshape(1, D))

    segsum = _make_segsum_sc()
    x0 = jnp.stack([node_feat[:, :HD], node_feat[:, HD:]])
    p1 = segsum(x0, src_r, dst_r, zeros)
    x1 = _layer_call(x0, p1, ew, gc_W[0])
    p2 = segsum(x1, src_r, dst_r, zeros)
    out = _layer2_call(x1, p2, ew, gc_W[1], lin_W)
    return out


# speedup vs baseline: 1.0515x; 1.0515x over previous
import functools

import jax
import jax.numpy as jnp
from jax import lax
from jax.experimental import pallas as pl
from jax.experimental.pallas import tpu as pltpu
from jax.experimental.pallas import tpu_sc as plsc

N = 10000
E = 320000
D = 128
HD = 64
R = 4
EC = 12
ECP = 16

NC = 2
NS = 16
NW = NC * NS
CH = 125
NCH = 80
EPT = NCH * CH
NACC = 10112
ZR = NACC // NS
BLK = 1000


@functools.lru_cache(maxsize=None)
def _make_segsum_sc():
    mesh = plsc.VectorSubcoreMesh(core_axis_name="c", subcore_axis_name="s")

    @functools.partial(
        pl.kernel,
        mesh=mesh,
        compiler_params=pltpu.CompilerParams(use_tc_tiling_on_sc=False),
        out_type=jax.ShapeDtypeStruct((2, NC, NACC, HD), jnp.float32),
        scratch_types=[
            pltpu.VMEM((NCH, CH), jnp.int32),
            pltpu.VMEM((NCH, CH), jnp.int32),
            pltpu.VMEM((CH, HD), jnp.float32),
            pltpu.VMEM((CH, HD), jnp.float32),
            pltpu.VMEM_SHARED((NACC, HD), jnp.float32),
            pltpu.VMEM_SHARED((N, HD), jnp.float32),
            pltpu.SemaphoreType.DMA,
            pltpu.SemaphoreType.DMA,
        ],
    )
    def _segsum_sc(x_hbm, src_hbm, dst_hbm, zeros_hbm, out_hbm,
                   src_v, dst_v, rows0, rows1, acc, table, sem0, sem1):
        c = lax.axis_index("c")
        s = lax.axis_index("s")
        wid = s * NC + c

        with jax.named_scope("sc_idx"):
            pltpu.sync_copy(src_hbm.at[wid], src_v)
            pltpu.sync_copy(dst_hbm.at[wid], dst_v)

        for h in range(2):
            with jax.named_scope("sc_stage"):
                pltpu.sync_copy(zeros_hbm, acc.at[pl.ds(s * ZR, ZR)])

                @pl.when(s < 10)
                def _stage_table():
                    pltpu.sync_copy(x_hbm.at[h, pl.ds(s * 1000, 1000)],
                                    table.at[pl.ds(s * 1000, 1000)])

                plsc.subcore_barrier()

            last = NCH - 1
            with jax.named_scope("sc_mainloop"):
                pltpu.async_copy(table.at[src_v.at[0]], rows0, sem0)

                def body(i, carry):
                    j = 2 * i
                    pltpu.async_copy(
                        table.at[src_v.at[jnp.minimum(j + 1, last)]],
                        rows1, sem1)
                    pltpu.make_async_copy(
                        table.at[src_v.at[0]], rows0, sem0).wait()
                    pltpu.sync_copy(rows0, acc.at[dst_v.at[j]], add=True)
                    pltpu.async_copy(
                        table.at[src_v.at[jnp.minimum(j + 2, last)]],
                        rows0, sem0)
                    pltpu.make_async_copy(
                        table.at[src_v.at[0]], rows1, sem1).wait()
                    pltpu.sync_copy(rows1, acc.at[dst_v.at[j + 1]], add=True)
                    return carry

                lax.fori_loop(0, NCH // 2, body, 0)
                pltpu.make_async_copy(table.at[src_v.at[0]], rows0, sem0).wait()
                plsc.subcore_barrier()

            with jax.named_scope("sc_out"):
                pltpu.sync_copy(acc.at[pl.ds(s * ZR, ZR)],
                                out_hbm.at[h, c, pl.ds(s * ZR, ZR)])

    return _segsum_sc


def _ew_body(h_ref, src_row_ref, src_col_ref, dst_row_ref, w_ref,
             beta_ref, eps_ref, W_ref, b_ref, out_ref):
    h = h_ref[...]
    csrc = src_row_ref[...]
    csrc_col = src_col_ref[...]
    cdst = dst_row_ref[...]
    w = w_ref[...]
    seg = lax.broadcasted_iota(jnp.int32, (R, ECP), 0)
    ohs = (seg == csrc)
    ohd = (seg == cdst).astype(jnp.float32)
    for l in range(2):
        nrm = jnp.sqrt(jnp.sum(h * h, axis=1, keepdims=True))
        norm_h = h / (nrm + 1e-12)
        e = beta_ref[l] * w
        m = jnp.max(jnp.where(ohs, e, -1e30), axis=1, keepdims=True)
        m = jnp.where(m < -1e29, 0.0, m)
        m_pe = jnp.sum(jnp.where(ohs, m, 0.0), axis=0, keepdims=True)
        ex = jnp.exp(e - m_pe)
        ssum = jnp.sum(jnp.where(ohs, ex, 0.0), axis=1, keepdims=True)
        s_pe = jnp.sum(jnp.where(ohs, ssum, 0.0), axis=0, keepdims=True)
        p = ex / (s_pe + 1e-16)
        gath = jnp.zeros((ECP, D), jnp.float32)
        for r in range(R):
            gath = gath + jnp.where(csrc_col == r, 1.0, 0.0) * norm_h[r:r + 1, :]
        agg = jnp.dot(ohd * p, gath,
                      preferred_element_type=jnp.float32,
                      precision=lax.Precision.HIGHEST)
        h = (1.0 + eps_ref[l]) * h + agg
        h = jnp.maximum(h, 0.0)
    ew = jnp.dot(h[0:1, :], W_ref[...],
                 preferred_element_type=jnp.float32,
                 precision=lax.Precision.HIGHEST) + b_ref[...]
    out_ref[...] = ew


def _ew_call(cl_h, src_row, src_col, dst_row, w_row, beta, eps, W, b):
    return pl.pallas_call(
        _ew_body,
        out_shape=jax.ShapeDtypeStruct((1, D), jnp.float32),
        in_specs=[
            pl.BlockSpec((R, D), lambda: (0, 0)),
            pl.BlockSpec((1, ECP), lambda: (0, 0)),
            pl.BlockSpec((ECP, 1), lambda: (0, 0)),
            pl.BlockSpec((1, ECP), lambda: (0, 0)),
            pl.BlockSpec((1, ECP), lambda: (0, 0)),
            pl.BlockSpec(memory_space=pltpu.SMEM),
            pl.BlockSpec(memory_space=pltpu.SMEM),
            pl.BlockSpec((D, D), lambda: (0, 0)),
            pl.BlockSpec((1, D), lambda: (0, 0)),
        ],
        out_specs=pl.BlockSpec((1, D), lambda: (0, 0)),
    )(cl_h, src_row, src_col, dst_row, w_row, beta, eps, W, b)


def _layer_body(x_ref, p_ref, ew_ref, W_ref, out_ref):
    x = jnp.concatenate([x_ref[0], x_ref[1]], axis=1)
    agg = jnp.concatenate([p_ref[0, 0] + p_ref[0, 1],
                           p_ref[1, 0] + p_ref[1, 1]], axis=1) * ew_ref[...]
    t = jnp.tanh(
        jnp.dot(agg + x, W_ref[...],
                preferred_element_type=jnp.float32))
    out_ref[0] = t[:, :HD]
    out_ref[1] = t[:, HD:]


def _layer2_body(x_ref, p_ref, ew_ref, W_ref, lW_ref, out_ref):
    x = jnp.concatenate([x_ref[0], x_ref[1]], axis=1)
    agg = jnp.concatenate([p_ref[0, 0] + p_ref[0, 1],
                           p_ref[1, 0] + p_ref[1, 1]], axis=1) * ew_ref[...]
    t = jnp.tanh(
        jnp.dot(agg + x, W_ref[...],
                preferred_element_type=jnp.float32))
    out_ref[...] = jnp.dot(t, lW_ref[...],
                           preferred_element_type=jnp.float32)


def _common_specs():
    return [
        pl.BlockSpec((2, BLK, HD), lambda i: (0, i, 0)),
        pl.BlockSpec((2, NC, BLK, HD), lambda i: (0, 0, i, 0)),
        pl.BlockSpec((1, D), lambda i: (0, 0)),
        pl.BlockSpec((D, D), lambda i: (0, 0)),
    ]


def _layer_call(x, p, ew, W):
    return pl.pallas_call(
        _layer_body,
        grid=(N // BLK,),
        out_shape=jax.ShapeDtypeStruct((2, N, HD), jnp.float32),
        in_specs=_common_specs(),
        out_specs=pl.BlockSpec((2, BLK, HD), lambda i: (0, i, 0)),
    )(x, p, ew, W)


def _layer2_call(x, p, ew, W, lW):
    return pl.pallas_call(
        _layer2_body,
        grid=(N // BLK,),
        out_shape=jax.ShapeDtypeStruct((N, D), jnp.float32),
        in_specs=_common_specs() + [pl.BlockSpec((D, D), lambda i: (0, 0))],
        out_specs=pl.BlockSpec((BLK, D), lambda i: (i, 0)),
    )(x, p, ew, W, lW)


def kernel(node_feat, edge_index, cl_h, cl_edge_index, cl_edge_w,
           beta, eps, lin_e1_W, lin_e1_b, gc_W, lin_W):
    src_r = edge_index[0].reshape(NW, NCH, CH)
    dst_r = edge_index[1].reshape(NW, NCH, CH)
    zeros = jnp.zeros((ZR, HD), jnp.float32)

    cpad = ECP - EC
    src_row = jnp.concatenate(
        [cl_edge_index[0], jnp.full((cpad,), R + 1, jnp.int32)]).reshape(1, ECP)
    src_col = src_row.reshape(ECP, 1)
    dst_row = jnp.concatenate(
        [cl_edge_index[1], jnp.full((cpad,), R + 1, jnp.int32)]).reshape(1, ECP)
    w_row = jnp.concatenate(
        [cl_edge_w, jnp.zeros((cpad,), jnp.float32)]).reshape(1, ECP)

    ew = _ew_call(cl_h, src_row, src_col, dst_row, w_row,
                  beta, eps, lin_e1_W, lin_e1_b.reshape(1, D))

    segsum = _make_segsum_sc()
    x0 = jnp.stack([node_feat[:, :HD], node_feat[:, HD:]])
    p1 = segsum(x0, src_r, dst_r, zeros)
    x1 = _layer_call(x0, p1, ew, gc_W[0])
    p2 = segsum(x1, src_r, dst_r, zeros)
    out = _layer2_call(x1, p2, ew, gc_W[1], lin_W)
    return out


# speedup vs baseline: 1.3810x; 1.3133x over previous
import functools

import jax
import jax.numpy as jnp
from jax import lax
from jax.experimental import pallas as pl
from jax.experimental.pallas import tpu as pltpu
from jax.experimental.pallas import tpu_sc as plsc

N = 10000
E = 320000
D = 128
HD = 64
R = 4
EC = 12
ECP = 16

NC = 2
NS = 16
NW = NC * NS
CH = 125
NCH = 80
EPT = NCH * CH
NACC = 10112
ZR = NACC // NS
BLK = 1000


@functools.lru_cache(maxsize=None)
def _make_segsum_sc():
    mesh = plsc.VectorSubcoreMesh(core_axis_name="c", subcore_axis_name="s")

    @functools.partial(
        pl.kernel,
        mesh=mesh,
        compiler_params=pltpu.CompilerParams(use_tc_tiling_on_sc=False),
        out_type=jax.ShapeDtypeStruct((2, NC, NACC, HD), jnp.float32),
        scratch_types=[
            pltpu.VMEM((NCH, CH), jnp.int32),
            pltpu.VMEM((NCH, CH), jnp.int32),
            pltpu.VMEM((CH, HD), jnp.float32),
            pltpu.VMEM((CH, HD), jnp.float32),
            pltpu.VMEM((CH, HD), jnp.float32),
            pltpu.VMEM((CH, HD), jnp.float32),
            pltpu.VMEM_SHARED((NACC, HD), jnp.float32),
            pltpu.SemaphoreType.DMA,
            pltpu.SemaphoreType.DMA,
            pltpu.SemaphoreType.DMA,
            pltpu.SemaphoreType.DMA,
        ],
    )
    def _segsum_sc(x_hbm, src_hbm, dst_hbm, zeros_hbm, out_hbm,
                   src_v, dst_v, rows0, rows1, rows2, rows3, acc,
                   sem0, sem1, sem2, sem3):
        c = lax.axis_index("c")
        s = lax.axis_index("s")
        wid = s * NC + c
        rows = (rows0, rows1, rows2, rows3)
        sems = (sem0, sem1, sem2, sem3)

        with jax.named_scope("sc_idx"):
            pltpu.sync_copy(src_hbm.at[wid], src_v)
            pltpu.sync_copy(dst_hbm.at[wid], dst_v)

        for h in range(2):
            xh = x_hbm.at[h]
            with jax.named_scope("sc_stage"):
                pltpu.sync_copy(zeros_hbm, acc.at[pl.ds(s * ZR, ZR)])
                plsc.subcore_barrier()

            last = NCH - 1
            with jax.named_scope("sc_mainloop"):
                for b in range(3):
                    pltpu.async_copy(xh.at[src_v.at[b]], rows[b], sems[b])

                def body(i, carry):
                    for b in range(4):
                        j = 4 * i + b
                        nb = (b + 3) % 4
                        pltpu.make_async_copy(
                            xh.at[src_v.at[0]], rows[b], sems[b]).wait()
                        pltpu.async_copy(
                            xh.at[src_v.at[jnp.minimum(j + 3, last)]],
                            rows[nb], sems[nb])
                        pltpu.sync_copy(rows[b], acc.at[dst_v.at[j]],
                                        add=True)
                    return carry

                lax.fori_loop(0, NCH // 4, body, 0)
                for b in range(3):
                    pltpu.make_async_copy(
                        xh.at[src_v.at[0]], rows[b], sems[b]).wait()
                plsc.subcore_barrier()

            with jax.named_scope("sc_out"):
                pltpu.sync_copy(acc.at[pl.ds(s * ZR, ZR)],
                                out_hbm.at[h, c, pl.ds(s * ZR, ZR)])

    return _segsum_sc


def _ew_body(h_ref, src_row_ref, src_col_ref, dst_row_ref, w_ref,
             beta_ref, eps_ref, W_ref, b_ref, out_ref):
    h = h_ref[...]
    csrc = src_row_ref[...]
    csrc_col = src_col_ref[...]
    cdst = dst_row_ref[...]
    w = w_ref[...]
    seg = lax.broadcasted_iota(jnp.int32, (R, ECP), 0)
    ohs = (seg == csrc)
    ohd = (seg == cdst).astype(jnp.float32)
    for l in range(2):
        nrm = jnp.sqrt(jnp.sum(h * h, axis=1, keepdims=True))
        norm_h = h / (nrm + 1e-12)
        e = beta_ref[l] * w
        m = jnp.max(jnp.where(ohs, e, -1e30), axis=1, keepdims=True)
        m = jnp.where(m < -1e29, 0.0, m)
        m_pe = jnp.sum(jnp.where(ohs, m, 0.0), axis=0, keepdims=True)
        ex = jnp.exp(e - m_pe)
        ssum = jnp.sum(jnp.where(ohs, ex, 0.0), axis=1, keepdims=True)
        s_pe = jnp.sum(jnp.where(ohs, ssum, 0.0), axis=0, keepdims=True)
        p = ex / (s_pe + 1e-16)
        gath = jnp.zeros((ECP, D), jnp.float32)
        for r in range(R):
            gath = gath + jnp.where(csrc_col == r, 1.0, 0.0) * norm_h[r:r + 1, :]
        agg = jnp.dot(ohd * p, gath,
                      preferred_element_type=jnp.float32,
                      precision=lax.Precision.HIGHEST)
        h = (1.0 + eps_ref[l]) * h + agg
        h = jnp.maximum(h, 0.0)
    ew = jnp.dot(h[0:1, :], W_ref[...],
                 preferred_element_type=jnp.float32,
                 precision=lax.Precision.HIGHEST) + b_ref[...]
    out_ref[...] = ew


def _ew_call(cl_h, src_row, src_col, dst_row, w_row, beta, eps, W, b):
    return pl.pallas_call(
        _ew_body,
        out_shape=jax.ShapeDtypeStruct((1, D), jnp.float32),
        in_specs=[
            pl.BlockSpec((R, D), lambda: (0, 0)),
            pl.BlockSpec((1, ECP), lambda: (0, 0)),
            pl.BlockSpec((ECP, 1), lambda: (0, 0)),
            pl.BlockSpec((1, ECP), lambda: (0, 0)),
            pl.BlockSpec((1, ECP), lambda: (0, 0)),
            pl.BlockSpec(memory_space=pltpu.SMEM),
            pl.BlockSpec(memory_space=pltpu.SMEM),
            pl.BlockSpec((D, D), lambda: (0, 0)),
            pl.BlockSpec((1, D), lambda: (0, 0)),
        ],
        out_specs=pl.BlockSpec((1, D), lambda: (0, 0)),
    )(cl_h, src_row, src_col, dst_row, w_row, beta, eps, W, b)


def _layer_body(x_ref, p_ref, ew_ref, W_ref, out_ref):
    x = jnp.concatenate([x_ref[0], x_ref[1]], axis=1)
    agg = jnp.concatenate([p_ref[0, 0] + p_ref[0, 1],
                           p_ref[1, 0] + p_ref[1, 1]], axis=1) * ew_ref[...]
    t = jnp.tanh(
        jnp.dot(agg + x, W_ref[...],
                preferred_element_type=jnp.float32))
    out_ref[0] = t[:, :HD]
    out_ref[1] = t[:, HD:]


def _layer2_body(x_ref, p_ref, ew_ref, W_ref, lW_ref, out_ref):
    x = jnp.concatenate([x_ref[0], x_ref[1]], axis=1)
    agg = jnp.concatenate([p_ref[0, 0] + p_ref[0, 1],
                           p_ref[1, 0] + p_ref[1, 1]], axis=1) * ew_ref[...]
    t = jnp.tanh(
        jnp.dot(agg + x, W_ref[...],
                preferred_element_type=jnp.float32))
    out_ref[...] = jnp.dot(t, lW_ref[...],
                           preferred_element_type=jnp.float32)


def _common_specs():
    return [
        pl.BlockSpec((2, BLK, HD), lambda i: (0, i, 0)),
        pl.BlockSpec((2, NC, BLK, HD), lambda i: (0, 0, i, 0)),
        pl.BlockSpec((1, D), lambda i: (0, 0)),
        pl.BlockSpec((D, D), lambda i: (0, 0)),
    ]


def _layer_call(x, p, ew, W):
    return pl.pallas_call(
        _layer_body,
        grid=(N // BLK,),
        out_shape=jax.ShapeDtypeStruct((2, N, HD), jnp.float32),
        in_specs=_common_specs(),
        out_specs=pl.BlockSpec((2, BLK, HD), lambda i: (0, i, 0)),
    )(x, p, ew, W)


def _layer2_call(x, p, ew, W, lW):
    return pl.pallas_call(
        _layer2_body,
        grid=(N // BLK,),
        out_shape=jax.ShapeDtypeStruct((N, D), jnp.float32),
        in_specs=_common_specs() + [pl.BlockSpec((D, D), lambda i: (0, 0))],
        out_specs=pl.BlockSpec((BLK, D), lambda i: (i, 0)),
    )(x, p, ew, W, lW)


def kernel(node_feat, edge_index, cl_h, cl_edge_index, cl_edge_w,
           beta, eps, lin_e1_W, lin_e1_b, gc_W, lin_W):
    src_r = edge_index[0].reshape(NW, NCH, CH)
    dst_r = edge_index[1].reshape(NW, NCH, CH)
    zeros = jnp.zeros((ZR, HD), jnp.float32)

    cpad = ECP - EC
    src_row = jnp.concatenate(
        [cl_edge_index[0], jnp.full((cpad,), R + 1, jnp.int32)]).reshape(1, ECP)
    src_col = src_row.reshape(ECP, 1)
    dst_row = jnp.concatenate(
        [cl_edge_index[1], jnp.full((cpad,), R + 1, jnp.int32)]).reshape(1, ECP)
    w_row = jnp.concatenate(
        [cl_edge_w, jnp.zeros((cpad,), jnp.float32)]).reshape(1, ECP)

    ew = _ew_call(cl_h, src_row, src_col, dst_row, w_row,
                  beta, eps, lin_e1_W, lin_e1_b.reshape(1, D))

    segsum = _make_segsum_sc()
    x0 = jnp.stack([node_feat[:, :HD], node_feat[:, HD:]])
    p1 = segsum(x0, src_r, dst_r, zeros)
    x1 = _layer_call(x0, p1, ew, gc_W[0])
    p2 = segsum(x1, src_r, dst_r, zeros)
    out = _layer2_call(x1, p2, ew, gc_W[1], lin_W)
    return out
